# Initial kernel scaffold; baseline (speedup 1.0000x reference)
import jax
import jax.numpy as jnp
from jax.experimental import pallas as pl


def kernel(x_author, x_paper, edge_index_ap, edge_index_pa, proj_author_w, proj_author_b, proj_paper_w, proj_paper_b, att_src_ap, att_dst_ap, att_src_pa, att_dst_pa, k_lin_w, k_lin_b, q, lin_w, lin_b):
    raise NotImplementedError("write your pallas kernel here")


# speedup vs baseline: 65.6577x; 65.6577x over previous
import functools

import jax
import jax.numpy as jnp
from jax import lax
from jax.experimental import pallas as pl
from jax.experimental.pallas import tpu as pltpu
from jax.experimental.pallas import tpu_sc as plsc

N = 10000
E = 320000
D_H = 128
HEADS = 8
DIM = 16
D_OUT = 64
ROW = 144

_NC = 2
_NS = 16
_NW = _NC * _NS
_CHUNK = 128
_TOT_CHUNKS = E // _CHUNK
_BASE_CH = _TOT_CHUNKS // _NW
_EXTRA = _TOT_CHUNKS - _BASE_CH * _NW
_NPAD = 10240
_RPT = _NPAD // _NS

_NBLK = 10
_BLK = N // _NBLK


def _pre_body(xa_ref, xp_ref, wa_ref, ba_ref, wp_ref, bp_ref, atts_ref,
              attd_ref, s16_ref, hsrc_ref, adst_ref, cmat_ref,
              accs_ref, accd_ref):
    i = pl.program_id(0)
    hp = jnp.dot(xp_ref[...], wp_ref[...],
                 preferred_element_type=jnp.float32) + bp_ref[...]
    ha = jnp.dot(xa_ref[...], wa_ref[...],
                 preferred_element_type=jnp.float32) + ba_ref[...]
    a16s = jnp.dot(hp * atts_ref[...], s16_ref[...],
                   preferred_element_type=jnp.float32)
    a16d = jnp.dot(ha * attd_ref[...], s16_ref[...],
                   preferred_element_type=jnp.float32)
    hsrc_ref[...] = jnp.concatenate([hp, a16s], axis=1)
    adst_ref[...] = a16d
    bs = jnp.broadcast_to(jnp.max(a16s, axis=0, keepdims=True), (8, 16))
    bd = jnp.broadcast_to(jnp.max(a16d, axis=0, keepdims=True), (8, 16))

    @pl.when(i == 0)
    def _():
        accs_ref[...] = bs
        accd_ref[...] = bd

    @pl.when(i > 0)
    def _():
        accs_ref[...] = jnp.maximum(accs_ref[...], bs)
        accd_ref[...] = jnp.maximum(accd_ref[...], bd)

    @pl.when(i == _NBLK - 1)
    def _():
        c = accs_ref[...] + accd_ref[...]
        cmat_ref[...] = jnp.maximum(c, 0.2 * c)


def _pre_call(x_author, x_paper, wa, ba, wp, bp, atts, attd, s16):
    return pl.pallas_call(
        _pre_body,
        grid=(_NBLK,),
        in_specs=[
            pl.BlockSpec((_BLK, D_H), lambda i: (i, 0)),
            pl.BlockSpec((_BLK, D_H), lambda i: (i, 0)),
            pl.BlockSpec((D_H, D_H), lambda i: (0, 0)),
            pl.BlockSpec((1, D_H), lambda i: (0, 0)),
            pl.BlockSpec((D_H, D_H), lambda i: (0, 0)),
            pl.BlockSpec((1, D_H), lambda i: (0, 0)),
            pl.BlockSpec((1, D_H), lambda i: (0, 0)),
            pl.BlockSpec((1, D_H), lambda i: (0, 0)),
            pl.BlockSpec((D_H, 16), lambda i: (0, 0)),
        ],
        out_specs=[
            pl.BlockSpec((_BLK, ROW), lambda i: (i, 0)),
            pl.BlockSpec((_BLK, 16), lambda i: (i, 0)),
            pl.BlockSpec((8, 16), lambda i: (0, 0)),
        ],
        out_shape=[
            jax.ShapeDtypeStruct((N, ROW), jnp.float32),
            jax.ShapeDtypeStruct((N, 16), jnp.float32),
            jax.ShapeDtypeStruct((8, 16), jnp.float32),
        ],
        scratch_shapes=[
            pltpu.VMEM((8, 16), jnp.float32),
            pltpu.VMEM((8, 16), jnp.float32),
        ],
    )(x_author, x_paper, wa, ba, wp, bp, atts, attd, s16)


def _sc_body(hsrc, adst_t, src_e, dst_e, cmat, out,
             accum, sidx, didx, rows, adstv, cbuf):
    cid = lax.axis_index("c")
    sid = lax.axis_index("s")
    w = sid * _NC + cid

    z16 = jnp.zeros((16,), jnp.float32)

    def _zr(r, _):
        for j in range(ROW // 16):
            rows[r, pl.ds(16 * j, 16)] = z16
        return 0

    lax.fori_loop(0, _CHUNK, _zr, 0)
    rbase = sid * _RPT
    for k in range(_RPT // _CHUNK):
        pltpu.sync_copy(rows, accum.at[pl.ds(rbase + k * _CHUNK, _CHUNK)])
    plsc.subcore_barrier()

    pltpu.sync_copy(cmat, cbuf)
    cv = cbuf[0]

    nch = _BASE_CH + jnp.where(w < _EXTRA, 1, 0)
    cstart = _BASE_CH * w + jnp.minimum(w, _EXTRA)

    def _chunk(i, _):
        eb = (cstart + i) * _CHUNK
        pltpu.sync_copy(src_e.at[pl.ds(eb, _CHUNK)], sidx)
        pltpu.sync_copy(dst_e.at[pl.ds(eb, _CHUNK)], didx)
        pltpu.sync_copy(hsrc.at[sidx], rows)
        pltpu.sync_copy(adst_t.at[didx], adstv)

        def _edge(e, _):
            srow = rows[e, pl.ds(D_H, 16)]
            al = srow + adstv[e]
            al = jnp.maximum(al, 0.2 * al)
            exv = jnp.exp(al - cv)
            rows[e, pl.ds(D_H, 16)] = exv
            for j in range(HEADS):
                s = exv[j]
                rows[e, pl.ds(DIM * j, DIM)] = rows[e, pl.ds(DIM * j, DIM)] * s
            return 0

        lax.fori_loop(0, _CHUNK, _edge, 0)
        pltpu.sync_copy(rows, accum.at[didx], add=True)
        return 0

    lax.fori_loop(0, nch, _chunk, 0)
    plsc.subcore_barrier()

    for k in range(_RPT // _CHUNK):
        pltpu.sync_copy(accum.at[pl.ds(rbase + k * _CHUNK, _CHUNK)],
                        out.at[cid, pl.ds(rbase + k * _CHUNK, _CHUNK)])


_sc_call = pl.kernel(
    _sc_body,
    out_type=jax.ShapeDtypeStruct((_NC, _NPAD, ROW), jnp.float32),
    mesh=plsc.VectorSubcoreMesh(core_axis_name="c", subcore_axis_name="s"),
    scratch_types=[
        pltpu.VMEM_SHARED((_NPAD, ROW), jnp.float32),
        pltpu.VMEM((_CHUNK,), jnp.int32),
        pltpu.VMEM((_CHUNK,), jnp.int32),
        pltpu.VMEM((_CHUNK, ROW), jnp.float32),
        pltpu.VMEM((_CHUNK, 16), jnp.float32),
        pltpu.VMEM((8, 16), jnp.float32),
    ],
    compiler_params=pltpu.CompilerParams(use_tc_tiling_on_sc=False),
)


def _epi_body(p_ref, exp16_ref, lw_ref, lb_ref, o_ref):
    p0 = p_ref[0]
    p1 = p_ref[1]
    num = p0[:, :D_H] + p1[:, :D_H]
    den16 = p0[:, D_H:ROW] + p1[:, D_H:ROW]
    den = jnp.dot(den16, exp16_ref[...], preferred_element_type=jnp.float32)
    o = jnp.maximum(num / (den + 1e-16), 0.0)
    o_ref[...] = jnp.dot(o, lw_ref[...],
                         preferred_element_type=jnp.float32) + lb_ref[...]


def _epi_call(partials, exp16, lw, lb):
    return pl.pallas_call(
        _epi_body,
        grid=(_NBLK,),
        in_specs=[
            pl.BlockSpec((_NC, _BLK, ROW), lambda i: (0, i, 0)),
            pl.BlockSpec((16, D_H), lambda i: (0, 0)),
            pl.BlockSpec((D_H, D_OUT), lambda i: (0, 0)),
            pl.BlockSpec((1, D_OUT), lambda i: (0, 0)),
        ],
        out_specs=pl.BlockSpec((_BLK, D_OUT), lambda i: (i, 0)),
        out_shape=jax.ShapeDtypeStruct((N, D_OUT), jnp.float32),
    )(partials, exp16, lw, lb)


def kernel(x_author, x_paper, edge_index_ap, edge_index_pa,
           proj_author_w, proj_author_b, proj_paper_w, proj_paper_b,
           att_src_ap, att_dst_ap, att_src_pa, att_dst_pa,
           k_lin_w, k_lin_b, q, lin_w, lin_b):
    src32 = edge_index_pa[0].astype(jnp.int32)
    dst32 = edge_index_pa[1].astype(jnp.int32)
    atts = att_src_pa.reshape(1, D_H)
    attd = att_dst_pa.reshape(1, D_H)
    s16 = (jnp.arange(D_H)[:, None] // DIM ==
           jnp.arange(16)[None, :]).astype(jnp.float32)
    exp16 = s16.T
    hsrc_tab, adst_tab, cmat = _pre_call(
        x_author, x_paper, proj_author_w, proj_author_b.reshape(1, D_H),
        proj_paper_w, proj_paper_b.reshape(1, D_H), atts, attd, s16)
    partials = _sc_call(hsrc_tab, adst_tab, src32, dst32, cmat)
    return _epi_call(partials, exp16, lin_w, lin_b.reshape(1, D_OUT))


# speedup vs baseline: 79.4826x; 1.2106x over previous
import functools

import jax
import jax.numpy as jnp
from jax import lax
from jax.experimental import pallas as pl
from jax.experimental.pallas import tpu as pltpu
from jax.experimental.pallas import tpu_sc as plsc

N = 10000
E = 320000
D_H = 128
HEADS = 8
DIM = 16
D_OUT = 64
ROW = 144

_NC = 2
_NS = 16
_NW = _NC * _NS
_CHUNK = 64
_TOT_CHUNKS = E // _CHUNK
_BASE_CH = _TOT_CHUNKS // _NW
_EXTRA = _TOT_CHUNKS - _BASE_CH * _NW
_EXTRA_CH = 1 if _EXTRA else 0
_NPAD = 10240
_RPT = _NPAD // _NS

_NBLK = 10
_BLK = N // _NBLK


def _pre_body(xa_ref, xp_ref, wa_ref, ba_ref, wp_ref, bp_ref, atts_ref,
              attd_ref, s16_ref, hsrc_ref, adst_ref, cmat_ref,
              accs_ref, accd_ref):
    i = pl.program_id(0)
    hp = jnp.dot(xp_ref[...], wp_ref[...],
                 preferred_element_type=jnp.float32) + bp_ref[...]
    ha = jnp.dot(xa_ref[...], wa_ref[...],
                 preferred_element_type=jnp.float32) + ba_ref[...]
    a16s = jnp.dot(hp * atts_ref[...], s16_ref[...],
                   preferred_element_type=jnp.float32)
    a16d = jnp.dot(ha * attd_ref[...], s16_ref[...],
                   preferred_element_type=jnp.float32)
    hsrc_ref[...] = jnp.concatenate([hp, a16s], axis=1)
    adst_ref[...] = a16d
    bs = jnp.broadcast_to(jnp.max(a16s, axis=0, keepdims=True), (8, 16))
    bd = jnp.broadcast_to(jnp.max(a16d, axis=0, keepdims=True), (8, 16))

    @pl.when(i == 0)
    def _():
        accs_ref[...] = bs
        accd_ref[...] = bd

    @pl.when(i > 0)
    def _():
        accs_ref[...] = jnp.maximum(accs_ref[...], bs)
        accd_ref[...] = jnp.maximum(accd_ref[...], bd)

    @pl.when(i == _NBLK - 1)
    def _():
        c = accs_ref[...] + accd_ref[...]
        cmat_ref[...] = jnp.maximum(c, 0.2 * c)


def _pre_call(x_author, x_paper, wa, ba, wp, bp, atts, attd, s16):
    return pl.pallas_call(
        _pre_body,
        grid=(_NBLK,),
        in_specs=[
            pl.BlockSpec((_BLK, D_H), lambda i: (i, 0)),
            pl.BlockSpec((_BLK, D_H), lambda i: (i, 0)),
            pl.BlockSpec((D_H, D_H), lambda i: (0, 0)),
            pl.BlockSpec((1, D_H), lambda i: (0, 0)),
            pl.BlockSpec((D_H, D_H), lambda i: (0, 0)),
            pl.BlockSpec((1, D_H), lambda i: (0, 0)),
            pl.BlockSpec((1, D_H), lambda i: (0, 0)),
            pl.BlockSpec((1, D_H), lambda i: (0, 0)),
            pl.BlockSpec((D_H, 16), lambda i: (0, 0)),
        ],
        out_specs=[
            pl.BlockSpec((_BLK, ROW), lambda i: (i, 0)),
            pl.BlockSpec((_BLK, 16), lambda i: (i, 0)),
            pl.BlockSpec((8, 16), lambda i: (0, 0)),
        ],
        out_shape=[
            jax.ShapeDtypeStruct((N, ROW), jnp.float32),
            jax.ShapeDtypeStruct((N, 16), jnp.float32),
            jax.ShapeDtypeStruct((8, 16), jnp.float32),
        ],
        scratch_shapes=[
            pltpu.VMEM((8, 16), jnp.float32),
            pltpu.VMEM((8, 16), jnp.float32),
        ],
    )(x_author, x_paper, wa, ba, wp, bp, atts, attd, s16)


def _sc_body(hsrc, adst_t, e2d, cmat, out,
             accum, sd, dxs, rows, adstv, cbuf, isem, gsem, ssem):
    cid = lax.axis_index("c")
    sid = lax.axis_index("s")
    w = sid * _NC + cid

    z16 = jnp.zeros((16,), jnp.float32)
    r0 = rows.at[0]

    def _zr(r, _):
        for j in range(ROW // 16):
            r0[r, pl.ds(16 * j, 16)] = z16
        return 0

    lax.fori_loop(0, _CHUNK, _zr, 0)
    rbase = sid * _RPT
    for k in range(_RPT // _CHUNK):
        pltpu.sync_copy(r0, accum.at[pl.ds(rbase + k * _CHUNK, _CHUNK)])
    plsc.subcore_barrier()

    pltpu.sync_copy(cmat, cbuf)
    cv = cbuf[0]

    cnt = _BASE_CH + jnp.where(w < _EXTRA, 1, 0)
    cstart = _BASE_CH * w + jnp.minimum(w, _EXTRA)

    def _fire_idx(c, j):
        pltpu.async_copy(e2d.at[cstart + c], sd.at[j], isem.at[j])

    def _fire_gather(c, j):
        pltpu.async_copy(hsrc.at[sd.at[j, 0]], rows.at[j], gsem.at[j])
        pltpu.async_copy(adst_t.at[sd.at[j, 1]], adstv.at[j], gsem.at[j])

    def _wait_gather(j):
        pltpu.make_async_copy(hsrc.at[sd.at[j, 0]], rows.at[j],
                              gsem.at[j]).wait()
        pltpu.make_async_copy(adst_t.at[sd.at[j, 1]], adstv.at[j],
                              gsem.at[j]).wait()

    def _wait_scatter(j):
        pltpu.make_async_copy(rows.at[j], accum.at[dxs.at[j]],
                              ssem.at[j]).wait()

    pltpu.sync_copy(e2d.at[cstart], sd.at[0])
    _fire_gather(0, 0)
    _fire_idx(1, 1)
    _fire_idx(2, 2)

    def _triple(t, _):
        for j in range(3):
            c = 3 * t + j
            n = (j + 1) % 3

            @pl.when(c < cnt)
            def _():
                _wait_gather(j)
                for k in range(_CHUNK // 16):
                    dxs[j, pl.ds(16 * k, 16)] = sd[j, 1, pl.ds(16 * k, 16)]

                @pl.when(c + 3 < cnt)
                def _():
                    _fire_idx(c + 3, j)

                rj = rows.at[j]
                aj = adstv.at[j]

                def _edge(e, _):
                    srow = rj[e, pl.ds(D_H, 16)]
                    al = srow + aj[e]
                    al = jnp.maximum(al, 0.2 * al)
                    exv = jnp.exp(al - cv)
                    rj[e, pl.ds(D_H, 16)] = exv
                    for h in range(HEADS):
                        s = exv[h]
                        rj[e, pl.ds(DIM * h, DIM)] = (
                            rj[e, pl.ds(DIM * h, DIM)] * s)
                    return 0

                lax.fori_loop(0, _CHUNK, _edge, 0, unroll=2)
                pltpu.async_copy(rows.at[j], accum.at[dxs.at[j]],
                                 ssem.at[j], add=True)

            @pl.when(c + 1 < cnt)
            def _():
                pltpu.make_async_copy(e2d.at[cstart + c + 1], sd.at[n],
                                      isem.at[n]).wait()

                @pl.when(c >= 2)
                def _():
                    _wait_scatter(n)

                _fire_gather(c + 1, n)
        return 0

    lax.fori_loop(0, (_BASE_CH + _EXTRA_CH + 2) // 3, _triple, 0)
    for j in range(3):
        _wait_scatter(j)
    plsc.subcore_barrier()

    for k in range(_RPT // _CHUNK):
        pltpu.sync_copy(accum.at[pl.ds(rbase + k * _CHUNK, _CHUNK)],
                        out.at[cid, pl.ds(rbase + k * _CHUNK, _CHUNK)])


_sc_call = pl.kernel(
    _sc_body,
    out_type=jax.ShapeDtypeStruct((_NC, _NPAD, ROW), jnp.float32),
    mesh=plsc.VectorSubcoreMesh(core_axis_name="c", subcore_axis_name="s"),
    scratch_types=[
        pltpu.VMEM_SHARED((_NPAD, ROW), jnp.float32),
        pltpu.VMEM((3, 2, _CHUNK), jnp.int32),
        pltpu.VMEM((3, _CHUNK), jnp.int32),
        pltpu.VMEM((3, _CHUNK, ROW), jnp.float32),
        pltpu.VMEM((3, _CHUNK, 16), jnp.float32),
        pltpu.VMEM((8, 16), jnp.float32),
        pltpu.SemaphoreType.DMA((3,)),
        pltpu.SemaphoreType.DMA((3,)),
        pltpu.SemaphoreType.DMA((3,)),
    ],
    compiler_params=pltpu.CompilerParams(use_tc_tiling_on_sc=False),
)


def _epi_body(p_ref, exp16_ref, lw_ref, lb_ref, o_ref):
    p0 = p_ref[0]
    p1 = p_ref[1]
    num = p0[:, :D_H] + p1[:, :D_H]
    den16 = p0[:, D_H:ROW] + p1[:, D_H:ROW]
    den = jnp.dot(den16, exp16_ref[...], preferred_element_type=jnp.float32)
    o = jnp.maximum(num / (den + 1e-16), 0.0)
    o_ref[...] = jnp.dot(o, lw_ref[...],
                         preferred_element_type=jnp.float32) + lb_ref[...]


def _epi_call(partials, exp16, lw, lb):
    return pl.pallas_call(
        _epi_body,
        grid=(_NBLK,),
        in_specs=[
            pl.BlockSpec((_NC, _BLK, ROW), lambda i: (0, i, 0)),
            pl.BlockSpec((16, D_H), lambda i: (0, 0)),
            pl.BlockSpec((D_H, D_OUT), lambda i: (0, 0)),
            pl.BlockSpec((1, D_OUT), lambda i: (0, 0)),
        ],
        out_specs=pl.BlockSpec((_BLK, D_OUT), lambda i: (i, 0)),
        out_shape=jax.ShapeDtypeStruct((N, D_OUT), jnp.float32),
    )(partials, exp16, lw, lb)


def kernel(x_author, x_paper, edge_index_ap, edge_index_pa,
           proj_author_w, proj_author_b, proj_paper_w, proj_paper_b,
           att_src_ap, att_dst_ap, att_src_pa, att_dst_pa,
           k_lin_w, k_lin_b, q, lin_w, lin_b):
    src32 = edge_index_pa[0].astype(jnp.int32).reshape(_TOT_CHUNKS, _CHUNK)
    dst32 = edge_index_pa[1].astype(jnp.int32).reshape(_TOT_CHUNKS, _CHUNK)
    e2d = jnp.stack([src32, dst32], axis=1)
    atts = att_src_pa.reshape(1, D_H)
    attd = att_dst_pa.reshape(1, D_H)
    s16 = (jnp.arange(D_H)[:, None] // DIM ==
           jnp.arange(16)[None, :]).astype(jnp.float32)
    exp16 = s16.T
    hsrc_tab, adst_tab, cmat = _pre_call(
        x_author, x_paper, proj_author_w, proj_author_b.reshape(1, D_H),
        proj_paper_w, proj_paper_b.reshape(1, D_H), atts, attd, s16)
    partials = _sc_call(hsrc_tab, adst_tab, e2d, cmat)
    return _epi_call(partials, exp16, lin_w, lin_b.reshape(1, D_OUT))


# speedup vs baseline: 81.4031x; 1.0242x over previous
import functools

import jax
import jax.numpy as jnp
from jax import lax
from jax.experimental import pallas as pl
from jax.experimental.pallas import tpu as pltpu
from jax.experimental.pallas import tpu_sc as plsc

N = 10000
E = 320000
D_H = 128
HEADS = 8
DIM = 16
D_OUT = 64
ROW = 144

_NC = 2
_NS = 16
_NW = _NC * _NS
_CHUNK = 64
_TOT_CHUNKS = E // _CHUNK
_BASE_CH = _TOT_CHUNKS // _NW
_EXTRA = _TOT_CHUNKS - _BASE_CH * _NW
_EXTRA_CH = 1 if _EXTRA else 0
_NPAD = 10240
_RPT = _NPAD // _NS

_NBLK = 10
_BLK = N // _NBLK


def _pre_body(xa_ref, xp_ref, wa_ref, ba_ref, wp_ref, bp_ref, atts_ref,
              attd_ref, s16_ref, hsrc_ref, adst_ref, cmat_ref,
              accs_ref, accd_ref):
    i = pl.program_id(0)
    hp = jnp.dot(xp_ref[...], wp_ref[...],
                 preferred_element_type=jnp.float32) + bp_ref[...]
    ha = jnp.dot(xa_ref[...], wa_ref[...],
                 preferred_element_type=jnp.float32) + ba_ref[...]
    a16s = jnp.dot(hp * atts_ref[...], s16_ref[...],
                   preferred_element_type=jnp.float32)
    a16d = jnp.dot(ha * attd_ref[...], s16_ref[...],
                   preferred_element_type=jnp.float32)
    hsrc_ref[...] = jnp.concatenate([hp, a16s], axis=1)
    adst_ref[...] = a16d
    bs = jnp.broadcast_to(jnp.max(a16s, axis=0, keepdims=True), (8, 16))
    bd = jnp.broadcast_to(jnp.max(a16d, axis=0, keepdims=True), (8, 16))

    @pl.when(i == 0)
    def _():
        accs_ref[...] = bs
        accd_ref[...] = bd

    @pl.when(i > 0)
    def _():
        accs_ref[...] = jnp.maximum(accs_ref[...], bs)
        accd_ref[...] = jnp.maximum(accd_ref[...], bd)

    @pl.when(i == _NBLK - 1)
    def _():
        c = accs_ref[...] + accd_ref[...]
        cmat_ref[...] = jnp.maximum(c, 0.2 * c)


def _pre_call(x_author, x_paper, wa, ba, wp, bp, atts, attd, s16):
    return pl.pallas_call(
        _pre_body,
        grid=(_NBLK,),
        in_specs=[
            pl.BlockSpec((_BLK, D_H), lambda i: (i, 0)),
            pl.BlockSpec((_BLK, D_H), lambda i: (i, 0)),
            pl.BlockSpec((D_H, D_H), lambda i: (0, 0)),
            pl.BlockSpec((1, D_H), lambda i: (0, 0)),
            pl.BlockSpec((D_H, D_H), lambda i: (0, 0)),
            pl.BlockSpec((1, D_H), lambda i: (0, 0)),
            pl.BlockSpec((1, D_H), lambda i: (0, 0)),
            pl.BlockSpec((1, D_H), lambda i: (0, 0)),
            pl.BlockSpec((D_H, 16), lambda i: (0, 0)),
        ],
        out_specs=[
            pl.BlockSpec((_BLK, ROW), lambda i: (i, 0)),
            pl.BlockSpec((_BLK, 16), lambda i: (i, 0)),
            pl.BlockSpec((8, 16), lambda i: (0, 0)),
        ],
        out_shape=[
            jax.ShapeDtypeStruct((N, ROW), jnp.float32),
            jax.ShapeDtypeStruct((N, 16), jnp.float32),
            jax.ShapeDtypeStruct((8, 16), jnp.float32),
        ],
        scratch_shapes=[
            pltpu.VMEM((8, 16), jnp.float32),
            pltpu.VMEM((8, 16), jnp.float32),
        ],
    )(x_author, x_paper, wa, ba, wp, bp, atts, attd, s16)


def _sc_body(hsrc, adst_t, e2d, cmat, out,
             accum, sd, dxs, rows, adstv, cbuf, isem, gsem, ssem):
    cid = lax.axis_index("c")
    sid = lax.axis_index("s")
    w = sid * _NC + cid

    z16 = jnp.zeros((16,), jnp.float32)
    r0 = rows.at[0]

    def _zr(r, _):
        for j in range(ROW // 16):
            r0[r, pl.ds(16 * j, 16)] = z16
        return 0

    lax.fori_loop(0, _CHUNK, _zr, 0)
    rbase = sid * _RPT
    for k in range(_RPT // _CHUNK):
        pltpu.sync_copy(r0, accum.at[pl.ds(rbase + k * _CHUNK, _CHUNK)])
    plsc.subcore_barrier()

    pltpu.sync_copy(cmat, cbuf)
    cv = cbuf[0]

    cnt = _BASE_CH + jnp.where(w < _EXTRA, 1, 0)
    cstart = _BASE_CH * w + jnp.minimum(w, _EXTRA)

    def _fire_idx(c, j):
        pltpu.async_copy(e2d.at[cstart + c], sd.at[j], isem.at[j])

    def _fire_gather(c, j):
        pltpu.async_copy(hsrc.at[sd.at[j, 0]], rows.at[j], gsem.at[j])
        pltpu.async_copy(adst_t.at[sd.at[j, 1]], adstv.at[j], gsem.at[j])

    def _wait_gather(j):
        pltpu.make_async_copy(hsrc.at[sd.at[j, 0]], rows.at[j],
                              gsem.at[j]).wait()
        pltpu.make_async_copy(adst_t.at[sd.at[j, 1]], adstv.at[j],
                              gsem.at[j]).wait()

    def _wait_scatter(j):
        pltpu.make_async_copy(rows.at[j], accum.at[dxs.at[j]],
                              ssem.at[j]).wait()

    pltpu.sync_copy(e2d.at[cstart], sd.at[0])
    _fire_gather(0, 0)
    _fire_idx(1, 1)
    _fire_idx(2, 2)

    def _triple(t, _):
        for j in range(3):
            c = 3 * t + j
            n = (j + 1) % 3

            @pl.when(c < cnt)
            def _():
                _wait_gather(j)
                for k in range(_CHUNK // 16):
                    dxs[j, pl.ds(16 * k, 16)] = sd[j, 1, pl.ds(16 * k, 16)]

                @pl.when(c + 3 < cnt)
                def _():
                    _fire_idx(c + 3, j)

                rj = rows.at[j]
                aj = adstv.at[j]

                def _exp_pass(e, _):
                    srow = rj[e, pl.ds(D_H, 16)]
                    al = srow + aj[e]
                    al = jnp.maximum(al, 0.2 * al)
                    rj[e, pl.ds(D_H, 16)] = jnp.exp(al - cv)
                    return 0

                def _scale_pass(e, _):
                    exv = rj[e, pl.ds(D_H, 16)]
                    for h in range(HEADS):
                        s = exv[h]
                        rj[e, pl.ds(DIM * h, DIM)] = (
                            rj[e, pl.ds(DIM * h, DIM)] * s)
                    return 0

                lax.fori_loop(0, _CHUNK, _exp_pass, 0, unroll=8)
                lax.fori_loop(0, _CHUNK, _scale_pass, 0, unroll=2)
                pltpu.async_copy(rows.at[j], accum.at[dxs.at[j]],
                                 ssem.at[j], add=True)

            @pl.when(c + 1 < cnt)
            def _():
                pltpu.make_async_copy(e2d.at[cstart + c + 1], sd.at[n],
                                      isem.at[n]).wait()

                @pl.when(c >= 2)
                def _():
                    _wait_scatter(n)

                _fire_gather(c + 1, n)
        return 0

    lax.fori_loop(0, (_BASE_CH + _EXTRA_CH + 2) // 3, _triple, 0)
    for j in range(3):
        _wait_scatter(j)
    plsc.subcore_barrier()

    for k in range(_RPT // _CHUNK):
        pltpu.sync_copy(accum.at[pl.ds(rbase + k * _CHUNK, _CHUNK)],
                        out.at[cid, pl.ds(rbase + k * _CHUNK, _CHUNK)])


_sc_call = pl.kernel(
    _sc_body,
    out_type=jax.ShapeDtypeStruct((_NC, _NPAD, ROW), jnp.float32),
    mesh=plsc.VectorSubcoreMesh(core_axis_name="c", subcore_axis_name="s"),
    scratch_types=[
        pltpu.VMEM_SHARED((_NPAD, ROW), jnp.float32),
        pltpu.VMEM((3, 2, _CHUNK), jnp.int32),
        pltpu.VMEM((3, _CHUNK), jnp.int32),
        pltpu.VMEM((3, _CHUNK, ROW), jnp.float32),
        pltpu.VMEM((3, _CHUNK, 16), jnp.float32),
        pltpu.VMEM((8, 16), jnp.float32),
        pltpu.SemaphoreType.DMA((3,)),
        pltpu.SemaphoreType.DMA((3,)),
        pltpu.SemaphoreType.DMA((3,)),
    ],
    compiler_params=pltpu.CompilerParams(use_tc_tiling_on_sc=False),
)


def _epi_body(p_ref, exp16_ref, lw_ref, lb_ref, o_ref):
    p0 = p_ref[0]
    p1 = p_ref[1]
    num = p0[:, :D_H] + p1[:, :D_H]
    den16 = p0[:, D_H:ROW] + p1[:, D_H:ROW]
    den = jnp.dot(den16, exp16_ref[...], preferred_element_type=jnp.float32)
    o = jnp.maximum(num / (den + 1e-16), 0.0)
    o_ref[...] = jnp.dot(o, lw_ref[...],
                         preferred_element_type=jnp.float32) + lb_ref[...]


def _epi_call(partials, exp16, lw, lb):
    return pl.pallas_call(
        _epi_body,
        grid=(_NBLK,),
        in_specs=[
            pl.BlockSpec((_NC, _BLK, ROW), lambda i: (0, i, 0)),
            pl.BlockSpec((16, D_H), lambda i: (0, 0)),
            pl.BlockSpec((D_H, D_OUT), lambda i: (0, 0)),
            pl.BlockSpec((1, D_OUT), lambda i: (0, 0)),
        ],
        out_specs=pl.BlockSpec((_BLK, D_OUT), lambda i: (i, 0)),
        out_shape=jax.ShapeDtypeStruct((N, D_OUT), jnp.float32),
    )(partials, exp16, lw, lb)


def kernel(x_author, x_paper, edge_index_ap, edge_index_pa,
           proj_author_w, proj_author_b, proj_paper_w, proj_paper_b,
           att_src_ap, att_dst_ap, att_src_pa, att_dst_pa,
           k_lin_w, k_lin_b, q, lin_w, lin_b):
    src32 = edge_index_pa[0].astype(jnp.int32).reshape(_TOT_CHUNKS, _CHUNK)
    dst32 = edge_index_pa[1].astype(jnp.int32).reshape(_TOT_CHUNKS, _CHUNK)
    e2d = jnp.stack([src32, dst32], axis=1)
    atts = att_src_pa.reshape(1, D_H)
    attd = att_dst_pa.reshape(1, D_H)
    s16 = (jnp.arange(D_H)[:, None] // DIM ==
           jnp.arange(16)[None, :]).astype(jnp.float32)
    exp16 = s16.T
    hsrc_tab, adst_tab, cmat = _pre_call(
        x_author, x_paper, proj_author_w, proj_author_b.reshape(1, D_H),
        proj_paper_w, proj_paper_b.reshape(1, D_H), atts, attd, s16)
    partials = _sc_call(hsrc_tab, adst_tab, e2d, cmat)
    return _epi_call(partials, exp16, lin_w, lin_b.reshape(1, D_OUT))


# speedup vs baseline: 122.2562x; 1.5019x over previous
import functools

import jax
import jax.numpy as jnp
from jax import lax
from jax.experimental import pallas as pl
from jax.experimental.pallas import tpu as pltpu
from jax.experimental.pallas import tpu_sc as plsc

N = 10000
E = 320000
D_H = 128
HEADS = 8
DIM = 16
D_OUT = 64
ROW = 144

_NC = 2
_NS = 16
_NW = _NC * _NS
_CHUNK = 64
_TOT_CHUNKS = E // _CHUNK
_BASE_CH = _TOT_CHUNKS // _NW
_EXTRA = _TOT_CHUNKS - _BASE_CH * _NW
_EXTRA_CH = 1 if _EXTRA else 0
_NPAD = 10240
_RPT = _NPAD // _NS

_NBLK = 10
_BLK = N // _NBLK


def _pre_body(xa_ref, xp_ref, wa_ref, ba_ref, wp_ref, bp_ref, atts_ref,
              attd_ref, s16_ref, hsrc_ref, adst_ref, cmat_ref,
              accs_ref, accd_ref):
    i = pl.program_id(0)
    hp = jnp.dot(xp_ref[...], wp_ref[...],
                 preferred_element_type=jnp.float32) + bp_ref[...]
    ha = jnp.dot(xa_ref[...], wa_ref[...],
                 preferred_element_type=jnp.float32) + ba_ref[...]
    a16s = jnp.dot(hp * atts_ref[...], s16_ref[...],
                   preferred_element_type=jnp.float32)
    a16d = jnp.dot(ha * attd_ref[...], s16_ref[...],
                   preferred_element_type=jnp.float32)
    hsrc_ref[...] = jnp.concatenate([hp, a16s], axis=1)
    adst_ref[...] = a16d
    bs = jnp.broadcast_to(jnp.max(a16s, axis=0, keepdims=True), (8, 16))
    bd = jnp.broadcast_to(jnp.max(a16d, axis=0, keepdims=True), (8, 16))

    @pl.when(i == 0)
    def _():
        accs_ref[...] = bs
        accd_ref[...] = bd

    @pl.when(i > 0)
    def _():
        accs_ref[...] = jnp.maximum(accs_ref[...], bs)
        accd_ref[...] = jnp.maximum(accd_ref[...], bd)

    @pl.when(i == _NBLK - 1)
    def _():
        c = accs_ref[...] + accd_ref[...]
        cmat_ref[...] = jnp.maximum(c, 0.2 * c)


def _pre_call(x_author, x_paper, wa, ba, wp, bp, atts, attd, s16):
    return pl.pallas_call(
        _pre_body,
        grid=(_NBLK,),
        in_specs=[
            pl.BlockSpec((_BLK, D_H), lambda i: (i, 0)),
            pl.BlockSpec((_BLK, D_H), lambda i: (i, 0)),
            pl.BlockSpec((D_H, D_H), lambda i: (0, 0)),
            pl.BlockSpec((1, D_H), lambda i: (0, 0)),
            pl.BlockSpec((D_H, D_H), lambda i: (0, 0)),
            pl.BlockSpec((1, D_H), lambda i: (0, 0)),
            pl.BlockSpec((1, D_H), lambda i: (0, 0)),
            pl.BlockSpec((1, D_H), lambda i: (0, 0)),
            pl.BlockSpec((D_H, 16), lambda i: (0, 0)),
        ],
        out_specs=[
            pl.BlockSpec((_BLK, ROW), lambda i: (i, 0)),
            pl.BlockSpec((_BLK, 16), lambda i: (i, 0)),
            pl.BlockSpec((8, 16), lambda i: (0, 0)),
        ],
        out_shape=[
            jax.ShapeDtypeStruct((N, ROW), jnp.float32),
            jax.ShapeDtypeStruct((N, 16), jnp.float32),
            jax.ShapeDtypeStruct((8, 16), jnp.float32),
        ],
        scratch_shapes=[
            pltpu.VMEM((8, 16), jnp.float32),
            pltpu.VMEM((8, 16), jnp.float32),
        ],
    )(x_author, x_paper, wa, ba, wp, bp, atts, attd, s16)


def _sc_body(hsrc, adst_t, e2d, cmat, out,
             accum, sd, dxs, rows, adstv, cbuf, isem, gsem, ssem):
    cid = lax.axis_index("c")
    sid = lax.axis_index("s")
    w = sid * _NC + cid

    z16 = jnp.zeros((16,), jnp.float32)
    r0 = rows.at[0]

    def _zr(r, _):
        for j in range(ROW // 16):
            r0[r, pl.ds(16 * j, 16)] = z16
        return 0

    lax.fori_loop(0, _CHUNK, _zr, 0)
    rbase = sid * _RPT
    for k in range(_RPT // _CHUNK):
        pltpu.sync_copy(r0, accum.at[pl.ds(rbase + k * _CHUNK, _CHUNK)])
    plsc.subcore_barrier()

    pltpu.sync_copy(cmat, cbuf)
    cv = cbuf[0]

    cnt = _BASE_CH + jnp.where(w < _EXTRA, 1, 0)
    cstart = _BASE_CH * w + jnp.minimum(w, _EXTRA)

    def _fire_idx(c, j):
        pltpu.async_copy(e2d.at[cstart + c], sd.at[j], isem.at[j])

    def _fire_gather(c, j):
        pltpu.async_copy(hsrc.at[sd.at[j, 0]], rows.at[j], gsem.at[j])
        pltpu.async_copy(adst_t.at[sd.at[j, 1]], adstv.at[j], gsem.at[j])

    def _wait_gather(j):
        pltpu.make_async_copy(hsrc.at[sd.at[j, 0]], rows.at[j],
                              gsem.at[j]).wait()
        pltpu.make_async_copy(adst_t.at[sd.at[j, 1]], adstv.at[j],
                              gsem.at[j]).wait()

    def _wait_scatter(j):
        pltpu.make_async_copy(rows.at[j], accum.at[dxs.at[j]],
                              ssem.at[j]).wait()

    pltpu.sync_copy(e2d.at[cstart], sd.at[0])
    _fire_gather(0, 0)
    _fire_idx(1, 1)
    _fire_idx(2, 2)

    def _triple(t, _):
        for j in range(3):
            c = 3 * t + j
            n = (j + 1) % 3

            @pl.when(c < cnt)
            def _():
                _wait_gather(j)
                for k in range(_CHUNK // 16):
                    dxs[j, pl.ds(16 * k, 16)] = sd[j, 1, pl.ds(16 * k, 16)]

                @pl.when(c + 3 < cnt)
                def _():
                    _fire_idx(c + 3, j)

                @pl.when(c + 1 < cnt)
                def _():
                    pltpu.make_async_copy(e2d.at[cstart + c + 1], sd.at[n],
                                          isem.at[n]).wait()

                    @pl.when(c >= 2)
                    def _():
                        _wait_scatter(n)

                    _fire_gather(c + 1, n)

                rj = rows.at[j]
                aj = adstv.at[j]

                def _exp_pass(e, _):
                    srow = rj[e, pl.ds(D_H, 16)]
                    al = srow + aj[e]
                    al = jnp.maximum(al, 0.2 * al)
                    rj[e, pl.ds(D_H, 16)] = jnp.exp(al - cv)
                    return 0

                def _scale_pass(e, _):
                    exv = rj[e, pl.ds(D_H, 16)]
                    for h in range(HEADS):
                        s = exv[h]
                        rj[e, pl.ds(DIM * h, DIM)] = (
                            rj[e, pl.ds(DIM * h, DIM)] * s)
                    return 0

                lax.fori_loop(0, _CHUNK, _exp_pass, 0, unroll=8)
                lax.fori_loop(0, _CHUNK, _scale_pass, 0, unroll=2)
                pltpu.async_copy(rows.at[j], accum.at[dxs.at[j]],
                                 ssem.at[j], add=True)
        return 0

    lax.fori_loop(0, (_BASE_CH + _EXTRA_CH + 2) // 3, _triple, 0)
    for j in range(3):
        _wait_scatter(j)
    plsc.subcore_barrier()

    for k in range(_RPT // _CHUNK):
        pltpu.sync_copy(accum.at[pl.ds(rbase + k * _CHUNK, _CHUNK)],
                        out.at[cid, pl.ds(rbase + k * _CHUNK, _CHUNK)])


_sc_call = pl.kernel(
    _sc_body,
    out_type=jax.ShapeDtypeStruct((_NC, _NPAD, ROW), jnp.float32),
    mesh=plsc.VectorSubcoreMesh(core_axis_name="c", subcore_axis_name="s"),
    scratch_types=[
        pltpu.VMEM_SHARED((_NPAD, ROW), jnp.float32),
        pltpu.VMEM((3, 2, _CHUNK), jnp.int32),
        pltpu.VMEM((3, _CHUNK), jnp.int32),
        pltpu.VMEM((3, _CHUNK, ROW), jnp.float32),
        pltpu.VMEM((3, _CHUNK, 16), jnp.float32),
        pltpu.VMEM((8, 16), jnp.float32),
        pltpu.SemaphoreType.DMA((3,)),
        pltpu.SemaphoreType.DMA((3,)),
        pltpu.SemaphoreType.DMA((3,)),
    ],
    compiler_params=pltpu.CompilerParams(use_tc_tiling_on_sc=False),
)


def _epi_body(p_ref, exp16_ref, lw_ref, lb_ref, o_ref):
    p0 = p_ref[0]
    p1 = p_ref[1]
    num = p0[:, :D_H] + p1[:, :D_H]
    den16 = p0[:, D_H:ROW] + p1[:, D_H:ROW]
    den = jnp.dot(den16, exp16_ref[...], preferred_element_type=jnp.float32)
    o = jnp.maximum(num / (den + 1e-16), 0.0)
    o_ref[...] = jnp.dot(o, lw_ref[...],
                         preferred_element_type=jnp.float32) + lb_ref[...]


def _epi_call(partials, exp16, lw, lb):
    return pl.pallas_call(
        _epi_body,
        grid=(_NBLK,),
        in_specs=[
            pl.BlockSpec((_NC, _BLK, ROW), lambda i: (0, i, 0)),
            pl.BlockSpec((16, D_H), lambda i: (0, 0)),
            pl.BlockSpec((D_H, D_OUT), lambda i: (0, 0)),
            pl.BlockSpec((1, D_OUT), lambda i: (0, 0)),
        ],
        out_specs=pl.BlockSpec((_BLK, D_OUT), lambda i: (i, 0)),
        out_shape=jax.ShapeDtypeStruct((N, D_OUT), jnp.float32),
    )(partials, exp16, lw, lb)


def kernel(x_author, x_paper, edge_index_ap, edge_index_pa,
           proj_author_w, proj_author_b, proj_paper_w, proj_paper_b,
           att_src_ap, att_dst_ap, att_src_pa, att_dst_pa,
           k_lin_w, k_lin_b, q, lin_w, lin_b):
    src32 = edge_index_pa[0].astype(jnp.int32).reshape(_TOT_CHUNKS, _CHUNK)
    dst32 = edge_index_pa[1].astype(jnp.int32).reshape(_TOT_CHUNKS, _CHUNK)
    e2d = jnp.stack([src32, dst32], axis=1)
    atts = att_src_pa.reshape(1, D_H)
    attd = att_dst_pa.reshape(1, D_H)
    s16 = (jnp.arange(D_H)[:, None] // DIM ==
           jnp.arange(16)[None, :]).astype(jnp.float32)
    exp16 = s16.T
    hsrc_tab, adst_tab, cmat = _pre_call(
        x_author, x_paper, proj_author_w, proj_author_b.reshape(1, D_H),
        proj_paper_w, proj_paper_b.reshape(1, D_H), atts, attd, s16)
    partials = _sc_call(hsrc_tab, adst_tab, e2d, cmat)
    return _epi_call(partials, exp16, lin_w, lin_b.reshape(1, D_OUT))


# speedup vs baseline: 123.1806x; 1.0076x over previous
import functools

import jax
import jax.numpy as jnp
from jax import lax
from jax.experimental import pallas as pl
from jax.experimental.pallas import tpu as pltpu
from jax.experimental.pallas import tpu_sc as plsc

N = 10000
E = 320000
D_H = 128
HEADS = 8
DIM = 16
D_OUT = 64
ROW = 144

_NC = 2
_NS = 16
_NW = _NC * _NS
_CHUNK = 64
_TOT_CHUNKS = E // _CHUNK
_BASE_CH = _TOT_CHUNKS // _NW
_EXTRA = _TOT_CHUNKS - _BASE_CH * _NW
_EXTRA_CH = 1 if _EXTRA else 0
_NPAD = 10240
_RPT = _NPAD // _NS

_NBLK = 10
_BLK = N // _NBLK


def _pre_body(xa_ref, xp_ref, wa_ref, ba_ref, wp_ref, bp_ref, atts_ref,
              attd_ref, s16_ref, hsrc_ref, adst_ref, cmat_ref,
              accs_ref, accd_ref):
    i = pl.program_id(0)
    hp = jnp.dot(xp_ref[...], wp_ref[...],
                 preferred_element_type=jnp.float32) + bp_ref[...]
    ha = jnp.dot(xa_ref[...], wa_ref[...],
                 preferred_element_type=jnp.float32) + ba_ref[...]
    a16s = jnp.dot(hp * atts_ref[...], s16_ref[...],
                   preferred_element_type=jnp.float32)
    a16d = jnp.dot(ha * attd_ref[...], s16_ref[...],
                   preferred_element_type=jnp.float32)
    hsrc_ref[...] = jnp.concatenate([hp, a16s], axis=1)
    adst_ref[...] = a16d
    bs = jnp.broadcast_to(jnp.max(a16s, axis=0, keepdims=True), (8, 16))
    bd = jnp.broadcast_to(jnp.max(a16d, axis=0, keepdims=True), (8, 16))

    @pl.when(i == 0)
    def _():
        accs_ref[...] = bs
        accd_ref[...] = bd

    @pl.when(i > 0)
    def _():
        accs_ref[...] = jnp.maximum(accs_ref[...], bs)
        accd_ref[...] = jnp.maximum(accd_ref[...], bd)

    @pl.when(i == _NBLK - 1)
    def _():
        c = accs_ref[...] + accd_ref[...]
        cmat_ref[...] = jnp.maximum(c, 0.2 * c)


def _pre_call(x_author, x_paper, wa, ba, wp, bp, atts, attd, s16):
    return pl.pallas_call(
        _pre_body,
        grid=(_NBLK,),
        in_specs=[
            pl.BlockSpec((_BLK, D_H), lambda i: (i, 0)),
            pl.BlockSpec((_BLK, D_H), lambda i: (i, 0)),
            pl.BlockSpec((D_H, D_H), lambda i: (0, 0)),
            pl.BlockSpec((1, D_H), lambda i: (0, 0)),
            pl.BlockSpec((D_H, D_H), lambda i: (0, 0)),
            pl.BlockSpec((1, D_H), lambda i: (0, 0)),
            pl.BlockSpec((1, D_H), lambda i: (0, 0)),
            pl.BlockSpec((1, D_H), lambda i: (0, 0)),
            pl.BlockSpec((D_H, 16), lambda i: (0, 0)),
        ],
        out_specs=[
            pl.BlockSpec((_BLK, ROW), lambda i: (i, 0)),
            pl.BlockSpec((_BLK, 16), lambda i: (i, 0)),
            pl.BlockSpec((8, 16), lambda i: (0, 0)),
        ],
        out_shape=[
            jax.ShapeDtypeStruct((N, ROW), jnp.float32),
            jax.ShapeDtypeStruct((N, 16), jnp.float32),
            jax.ShapeDtypeStruct((8, 16), jnp.float32),
        ],
        scratch_shapes=[
            pltpu.VMEM((8, 16), jnp.float32),
            pltpu.VMEM((8, 16), jnp.float32),
        ],
    )(x_author, x_paper, wa, ba, wp, bp, atts, attd, s16)


def _sc_body(hsrc, adst_t, e2d, cmat, out,
             accum, sd, dxs, rows, adstv, cbuf, isem, gsem, ssem):
    cid = lax.axis_index("c")
    sid = lax.axis_index("s")
    w = sid * _NC + cid

    z16 = jnp.zeros((16,), jnp.float32)
    r0 = rows.at[0]

    def _zr(r, _):
        for j in range(ROW // 16):
            r0[r, pl.ds(16 * j, 16)] = z16
        return 0

    lax.fori_loop(0, _CHUNK, _zr, 0)
    rbase = sid * _RPT
    for k in range(_RPT // _CHUNK):
        pltpu.async_copy(r0, accum.at[pl.ds(rbase + k * _CHUNK, _CHUNK)],
                         gsem.at[0])
    pltpu.sync_copy(cmat, cbuf)
    cv = cbuf[0]
    for k in range(_RPT // _CHUNK):
        pltpu.make_async_copy(r0, accum.at[pl.ds(rbase + k * _CHUNK, _CHUNK)],
                              gsem.at[0]).wait()
    plsc.subcore_barrier()

    cnt = _BASE_CH + jnp.where(w < _EXTRA, 1, 0)
    cstart = _BASE_CH * w + jnp.minimum(w, _EXTRA)

    def _fire_idx(c, j):
        pltpu.async_copy(e2d.at[cstart + c], sd.at[j], isem.at[j])

    def _fire_gather(c, j):
        pltpu.async_copy(hsrc.at[sd.at[j, 0]], rows.at[j], gsem.at[j])
        pltpu.async_copy(adst_t.at[sd.at[j, 1]], adstv.at[j], gsem.at[j])

    def _wait_gather(j):
        pltpu.make_async_copy(hsrc.at[sd.at[j, 0]], rows.at[j],
                              gsem.at[j]).wait()
        pltpu.make_async_copy(adst_t.at[sd.at[j, 1]], adstv.at[j],
                              gsem.at[j]).wait()

    def _wait_scatter(j):
        pltpu.make_async_copy(rows.at[j], accum.at[dxs.at[j]],
                              ssem.at[j]).wait()

    pltpu.sync_copy(e2d.at[cstart], sd.at[0])
    _fire_gather(0, 0)
    _fire_idx(1, 1)
    _fire_idx(2, 2)

    def _triple(t, _):
        for j in range(3):
            c = 3 * t + j
            n = (j + 1) % 3

            @pl.when(c < cnt)
            def _():
                _wait_gather(j)
                for k in range(_CHUNK // 16):
                    dxs[j, pl.ds(16 * k, 16)] = sd[j, 1, pl.ds(16 * k, 16)]

                @pl.when(c + 3 < cnt)
                def _():
                    _fire_idx(c + 3, j)

                @pl.when(c + 1 < cnt)
                def _():
                    pltpu.make_async_copy(e2d.at[cstart + c + 1], sd.at[n],
                                          isem.at[n]).wait()

                    @pl.when(c >= 2)
                    def _():
                        _wait_scatter(n)

                    _fire_gather(c + 1, n)

                rj = rows.at[j]
                aj = adstv.at[j]

                def _exp_pass(e, _):
                    srow = rj[e, pl.ds(D_H, 16)]
                    al = srow + aj[e]
                    al = jnp.maximum(al, 0.2 * al)
                    rj[e, pl.ds(D_H, 16)] = jnp.exp(al - cv)
                    return 0

                def _scale_pass(e, _):
                    exv = rj[e, pl.ds(D_H, 16)]
                    for h in range(HEADS):
                        s = exv[h]
                        rj[e, pl.ds(DIM * h, DIM)] = (
                            rj[e, pl.ds(DIM * h, DIM)] * s)
                    return 0

                lax.fori_loop(0, _CHUNK, _exp_pass, 0, unroll=8)
                lax.fori_loop(0, _CHUNK, _scale_pass, 0, unroll=2)
                pltpu.async_copy(rows.at[j], accum.at[dxs.at[j]],
                                 ssem.at[j], add=True)
        return 0

    lax.fori_loop(0, (_BASE_CH + _EXTRA_CH + 2) // 3, _triple, 0)
    for j in range(3):
        _wait_scatter(j)
    plsc.subcore_barrier()

    for k in range(_RPT // _CHUNK):
        pltpu.async_copy(accum.at[pl.ds(rbase + k * _CHUNK, _CHUNK)],
                         out.at[cid, pl.ds(rbase + k * _CHUNK, _CHUNK)],
                         gsem.at[0])
    for k in range(_RPT // _CHUNK):
        pltpu.make_async_copy(accum.at[pl.ds(rbase + k * _CHUNK, _CHUNK)],
                              out.at[cid, pl.ds(rbase + k * _CHUNK, _CHUNK)],
                              gsem.at[0]).wait()


_sc_call = pl.kernel(
    _sc_body,
    out_type=jax.ShapeDtypeStruct((_NC, _NPAD, ROW), jnp.float32),
    mesh=plsc.VectorSubcoreMesh(core_axis_name="c", subcore_axis_name="s"),
    scratch_types=[
        pltpu.VMEM_SHARED((_NPAD, ROW), jnp.float32),
        pltpu.VMEM((3, 2, _CHUNK), jnp.int32),
        pltpu.VMEM((3, _CHUNK), jnp.int32),
        pltpu.VMEM((3, _CHUNK, ROW), jnp.float32),
        pltpu.VMEM((3, _CHUNK, 16), jnp.float32),
        pltpu.VMEM((8, 16), jnp.float32),
        pltpu.SemaphoreType.DMA((3,)),
        pltpu.SemaphoreType.DMA((3,)),
        pltpu.SemaphoreType.DMA((3,)),
    ],
    compiler_params=pltpu.CompilerParams(use_tc_tiling_on_sc=False),
)


def _epi_body(p_ref, exp16_ref, lw_ref, lb_ref, o_ref):
    p0 = p_ref[0]
    p1 = p_ref[1]
    num = p0[:, :D_H] + p1[:, :D_H]
    den16 = p0[:, D_H:ROW] + p1[:, D_H:ROW]
    den = jnp.dot(den16, exp16_ref[...], preferred_element_type=jnp.float32)
    o = jnp.maximum(num / (den + 1e-16), 0.0)
    o_ref[...] = jnp.dot(o, lw_ref[...],
                         preferred_element_type=jnp.float32) + lb_ref[...]


def _epi_call(partials, exp16, lw, lb):
    return pl.pallas_call(
        _epi_body,
        grid=(_NBLK,),
        in_specs=[
            pl.BlockSpec((_NC, _BLK, ROW), lambda i: (0, i, 0)),
            pl.BlockSpec((16, D_H), lambda i: (0, 0)),
            pl.BlockSpec((D_H, D_OUT), lambda i: (0, 0)),
            pl.BlockSpec((1, D_OUT), lambda i: (0, 0)),
        ],
        out_specs=pl.BlockSpec((_BLK, D_OUT), lambda i: (i, 0)),
        out_shape=jax.ShapeDtypeStruct((N, D_OUT), jnp.float32),
    )(partials, exp16, lw, lb)


def kernel(x_author, x_paper, edge_index_ap, edge_index_pa,
           proj_author_w, proj_author_b, proj_paper_w, proj_paper_b,
           att_src_ap, att_dst_ap, att_src_pa, att_dst_pa,
           k_lin_w, k_lin_b, q, lin_w, lin_b):
    src32 = edge_index_pa[0].astype(jnp.int32).reshape(_TOT_CHUNKS, _CHUNK)
    dst32 = edge_index_pa[1].astype(jnp.int32).reshape(_TOT_CHUNKS, _CHUNK)
    e2d = jnp.stack([src32, dst32], axis=1)
    atts = att_src_pa.reshape(1, D_H)
    attd = att_dst_pa.reshape(1, D_H)
    s16 = (jnp.arange(D_H)[:, None] // DIM ==
           jnp.arange(16)[None, :]).astype(jnp.float32)
    exp16 = s16.T
    hsrc_tab, adst_tab, cmat = _pre_call(
        x_author, x_paper, proj_author_w, proj_author_b.reshape(1, D_H),
        proj_paper_w, proj_paper_b.reshape(1, D_H), atts, attd, s16)
    partials = _sc_call(hsrc_tab, adst_tab, e2d, cmat)
    return _epi_call(partials, exp16, lin_w, lin_b.reshape(1, D_OUT))
